# baseline (speedup 1.0000x reference)
import functools

import jax
import jax.numpy as jnp
from jax import lax
from jax.experimental import pallas as pl
from jax.experimental.pallas import tpu as pltpu
from jax.experimental.pallas import tpu_sc as plsc

_NC, _NS = 2, 16
_CHUNK = 800
_NORM_BLOCK = 2048


def _sc_gather(wv, idx4):
    num_idx = idx4.shape[0]
    dw = wv.shape[1]
    nw = _NC * _NS
    b_per_w = num_idx // nw
    mesh = plsc.VectorSubcoreMesh(core_axis_name="c", subcore_axis_name="s")

    @functools.partial(
        pl.kernel,
        mesh=mesh,
        out_type=jax.ShapeDtypeStruct((num_idx, dw), wv.dtype),
        scratch_types=[
            pltpu.VMEM((_CHUNK,), jnp.int32),
            pltpu.VMEM((_CHUNK, dw), jnp.float32),
            pltpu.SemaphoreType.DMA,
        ],
    )
    def gather_kernel(w_hbm, i_hbm, o_hbm, idx_v, rows_v, sem):
        wid = lax.axis_index("s") * _NC + lax.axis_index("c")
        base = wid * b_per_w

        @pl.loop(0, b_per_w, step=_CHUNK)
        def _(off):
            pltpu.sync_copy(i_hbm.at[pl.ds(base + off, _CHUNK)], idx_v)
            pltpu.async_copy(w_hbm.at[idx_v], rows_v, sem).wait()
            pltpu.sync_copy(rows_v, o_hbm.at[pl.ds(base + off, _CHUNK)])

    return gather_kernel(wv, idx4)


def _select_normalize(g, r4, d):
    n, dw = g.shape
    nsub = dw // d

    def body(g_ref, r_ref, o_ref):
        gb = g_ref[...]
        q = r_ref[...]
        acc = jnp.zeros((gb.shape[0], d), jnp.float32)
        for k in range(nsub):
            mask = jnp.where(q == k, 1.0, 0.0)
            acc = acc + mask * gb[:, k * d:(k + 1) * d]
        s = jnp.sum(acc * acc, axis=1, keepdims=True)
        o_ref[...] = acc / jnp.maximum(jnp.sqrt(s), 1e-12)

    return pl.pallas_call(
        body,
        grid=(n // _NORM_BLOCK,),
        in_specs=[
            pl.BlockSpec((_NORM_BLOCK, dw), lambda i: (i, 0)),
            pl.BlockSpec((_NORM_BLOCK, 1), lambda i: (i, 0)),
        ],
        out_specs=pl.BlockSpec((_NORM_BLOCK, d), lambda i: (i, 0)),
        out_shape=jax.ShapeDtypeStruct((n, d), jnp.float32),
    )(g, r4)


def kernel(x, weight):
    b, h = x.shape
    n, d = weight.shape
    num_idx = b * h
    idxf = x.reshape(num_idx).astype(jnp.int32)
    wv = weight.reshape(n // 4, 4 * d)
    idx4 = idxf // 4
    r4 = (idxf % 4).reshape(num_idx, 1)
    g = _sc_gather(wv, idx4)
    out = _select_normalize(g, r4, d)
    return out.reshape(b, h, d)
